# baseline (speedup 1.0000x reference)
import math

import jax
import jax.numpy as jnp
import numpy as np
from jax.experimental import pallas as pl
from jax.experimental.pallas import tpu as pltpu
from jax.experimental.shard_map import shard_map
from jax.sharding import Mesh, PartitionSpec as P


def _pab_kernel(qkv_ref, e_ref, w_ref, o_ref):
    ch = o_ref.shape[1]
    T = o_ref.shape[2]
    scale = 1.0 / math.sqrt(math.sqrt(ch))
    e = e_ref[0]
    dn_te = (((1,), (1,)), ((), ()))
    null = jax.lax.dot_general(
        w_ref[...], e, dn_te, preferred_element_type=jnp.float32)
    qe = (qkv_ref[0, 0:ch, :] + null[0:ch, :]) * scale
    ke = (qkv_ref[0, ch:2 * ch, :] + null[ch:2 * ch, :]) * scale
    ve = qkv_ref[0, 2 * ch:3 * ch, :] + null[2 * ch:3 * ch, :]
    logits = jax.lax.dot_general(
        qe, ke, (((0,), (0,)), ((), ())),
        preferred_element_type=jnp.float32)
    ew = jnp.exp(logits)
    ve_aug = jnp.concatenate(
        [ve.T, jnp.ones((T, 8), dtype=jnp.float32)], axis=1)
    a_aug = jax.lax.dot_general(
        ew, ve_aug, (((1,), (0,)), ((), ())),
        preferred_element_type=jnp.float32)
    a_t = a_aug[:, 0:ch] / a_aug[:, ch:ch + 1]
    o_ref[0] = a_t.T


def _run(qkv_r, cls_embedding, W_cls):
    B, width, T = qkv_r.shape
    ch = width // 3
    E = cls_embedding.shape[2]
    return pl.pallas_call(
        _pab_kernel,
        grid=(B,),
        in_specs=[
            pl.BlockSpec((1, 3 * ch, T), lambda b: (b, 0, 0)),
            pl.BlockSpec((1, T, E), lambda b: (b, 0, 0)),
            pl.BlockSpec((3 * ch, E), lambda b: (0, 0)),
        ],
        out_specs=pl.BlockSpec((1, ch, T), lambda b: (b, 0, 0)),
        out_shape=jax.ShapeDtypeStruct((B, ch, T), qkv_r.dtype),
    )(qkv_r, cls_embedding, W_cls)


def kernel(qkv, cls_embedding, W_cls):
    bs, width, T = qkv.shape
    n_heads = 16
    ch = width // (3 * n_heads)
    B = bs * n_heads
    qkv_r = qkv.reshape(B, 3 * ch, T)
    devs = jax.devices()
    if len(devs) >= 2 and B % 2 == 0:
        mesh = Mesh(np.asarray(devs[:2]), ("d",))
        out = shard_map(
            _run, mesh=mesh,
            in_specs=(P("d"), P("d"), P()),
            out_specs=P("d"),
            check_rep=False,
        )(qkv_r, cls_embedding, W_cls)
    else:
        out = _run(qkv_r, cls_embedding, W_cls)
    return out.reshape(bs, n_heads * ch, T)


# speedup vs baseline: 3.0548x; 3.0548x over previous
import math

import jax
import jax.numpy as jnp
from jax.experimental import pallas as pl
from jax.experimental.pallas import tpu as pltpu

_SPLIT = 1280

_C = (0.6931471805599453, 0.2402265069591007, 0.05550410866482158,
      0.009618129107628477, 0.0013333558146428443, 0.00015403530393381608)


def _exp2_poly(x):
    y = x * 1.4426950408889634
    k = jnp.floor(y + 0.5)
    f = y - k
    p = 1.0 + f * (_C[0] + f * (_C[1] + f * (_C[2] + f * (_C[3] + f * (_C[4] + f * _C[5])))))
    ki = jnp.clip(k, -126.0, 127.0).astype(jnp.int32)
    two_k = jax.lax.bitcast_convert_type((ki + 127) << 23, jnp.float32)
    return p * two_k


def _pab_kernel(qkv_ref, e_ref, w_ref, o_ref):
    ch = o_ref.shape[1]
    T = o_ref.shape[2]
    scale = 1.0 / math.sqrt(math.sqrt(ch))
    e = e_ref[0]
    dn_te = (((1,), (1,)), ((), ()))
    null = jax.lax.dot_general(
        w_ref[...], e, dn_te, preferred_element_type=jnp.float32)
    qe = (qkv_ref[0, 0:ch, :] + null[0:ch, :]) * scale
    ke = (qkv_ref[0, ch:2 * ch, :] + null[ch:2 * ch, :]) * scale
    ve = qkv_ref[0, 2 * ch:3 * ch, :] + null[2 * ch:3 * ch, :]
    logits = jax.lax.dot_general(
        qe, ke, (((0,), (0,)), ((), ())),
        preferred_element_type=jnp.float32)
    ew = jnp.exp(logits).astype(jnp.bfloat16)
    ve_aug = jnp.concatenate(
        [ve.T, jnp.ones((T, 8), dtype=jnp.float32)], axis=1)
    a_aug = jax.lax.dot_general(
        ew, ve_aug.astype(jnp.bfloat16), (((1,), (0,)), ((), ())),
        preferred_element_type=jnp.float32)
    a_t = a_aug[:, 0:ch] / a_aug[:, ch:ch + 1]
    o_ref[0] = a_t.T


def kernel(qkv, cls_embedding, W_cls):
    bs, width, T = qkv.shape
    n_heads = 16
    ch = width // (3 * n_heads)
    B = bs * n_heads
    E = cls_embedding.shape[2]
    qkv_r = qkv.reshape(B, 3 * ch, T)
    out = pl.pallas_call(
        _pab_kernel,
        grid=(B,),
        in_specs=[
            pl.BlockSpec((1, 3 * ch, T), lambda b: (b, 0, 0)),
            pl.BlockSpec((1, T, E), lambda b: (b, 0, 0)),
            pl.BlockSpec((3 * ch, E), lambda b: (0, 0)),
        ],
        out_specs=pl.BlockSpec((1, ch, T), lambda b: (b, 0, 0)),
        out_shape=jax.ShapeDtypeStruct((B, ch, T), qkv.dtype),
    )(qkv_r, cls_embedding, W_cls)
    return out.reshape(bs, n_heads * ch, T)


# speedup vs baseline: 3.2246x; 1.0556x over previous
import math

import jax
import jax.numpy as jnp
from jax.experimental import pallas as pl
from jax.experimental.pallas import tpu as pltpu


def _head(qkv_ref, e_ref, w_ref, o_ref, h):
    ch = o_ref.shape[1]
    T = o_ref.shape[2]
    scale = 1.0 / math.sqrt(math.sqrt(ch))
    e = e_ref[h]
    dn_te = (((1,), (1,)), ((), ()))
    null = jax.lax.dot_general(
        w_ref[...], e, dn_te, preferred_element_type=jnp.float32)
    qe = (qkv_ref[h, 0:ch, :] + null[0:ch, :]) * scale
    ke = (qkv_ref[h, ch:2 * ch, :] + null[ch:2 * ch, :]) * scale
    ve = qkv_ref[h, 2 * ch:3 * ch, :] + null[2 * ch:3 * ch, :]
    logits = jax.lax.dot_general(
        qe, ke, (((0,), (0,)), ((), ())),
        preferred_element_type=jnp.float32)
    ew = jnp.exp(logits).astype(jnp.bfloat16)
    ve_aug = jnp.concatenate(
        [ve.T, jnp.ones((T, 8), dtype=jnp.float32)], axis=1)
    a_aug = jax.lax.dot_general(
        ew, ve_aug.astype(jnp.bfloat16), (((1,), (0,)), ((), ())),
        preferred_element_type=jnp.float32)
    a_t = a_aug[:, 0:ch] / a_aug[:, ch:ch + 1]
    o_ref[h] = a_t.T


def _pab_kernel(qkv_ref, e_ref, w_ref, o_ref):
    _head(qkv_ref, e_ref, w_ref, o_ref, 0)
    _head(qkv_ref, e_ref, w_ref, o_ref, 1)


def kernel(qkv, cls_embedding, W_cls):
    bs, width, T = qkv.shape
    n_heads = 16
    ch = width // (3 * n_heads)
    B = bs * n_heads
    E = cls_embedding.shape[2]
    qkv_r = qkv.reshape(B, 3 * ch, T)
    out = pl.pallas_call(
        _pab_kernel,
        grid=(B // 2,),
        in_specs=[
            pl.BlockSpec((2, 3 * ch, T), lambda b: (b, 0, 0)),
            pl.BlockSpec((2, T, E), lambda b: (b, 0, 0)),
            pl.BlockSpec((3 * ch, E), lambda b: (0, 0)),
        ],
        out_specs=pl.BlockSpec((2, ch, T), lambda b: (b, 0, 0)),
        out_shape=jax.ShapeDtypeStruct((B, ch, T), qkv.dtype),
    )(qkv_r, cls_embedding, W_cls)
    return out.reshape(bs, n_heads * ch, T)


# speedup vs baseline: 3.2595x; 1.0108x over previous
import math

import jax
import jax.numpy as jnp
from jax.experimental import pallas as pl


def _head(qkv_ref, e_ref, w_ref, o_ref, h):
    ch = o_ref.shape[1]
    T = o_ref.shape[2]
    scale = 1.0 / math.sqrt(math.sqrt(ch))
    e = e_ref[h]
    dn_te = (((1,), (1,)), ((), ()))
    null = jax.lax.dot_general(
        w_ref[...], e, dn_te, preferred_element_type=jnp.float32)
    qe = (qkv_ref[h, 0:ch, :] + null[0:ch, :]) * scale
    ke = (qkv_ref[h, ch:2 * ch, :] + null[ch:2 * ch, :]) * scale
    ve = qkv_ref[h, 2 * ch:3 * ch, :] + null[2 * ch:3 * ch, :]
    logits = jax.lax.dot_general(
        qe, ke, (((0,), (0,)), ((), ())),
        preferred_element_type=jnp.float32)
    ew = jnp.exp(logits).astype(jnp.bfloat16)
    ve_aug = jnp.concatenate(
        [ve.T, jnp.ones((T, 8), dtype=jnp.float32)], axis=1)
    a_aug = jax.lax.dot_general(
        ew, ve_aug.astype(jnp.bfloat16), (((1,), (0,)), ((), ())),
        preferred_element_type=jnp.float32)
    a_t = a_aug[:, 0:ch] / a_aug[:, ch:ch + 1]
    o_ref[h] = a_t.T


def _pab_kernel(qkv_ref, e_ref, w_ref, o_ref):
    _head(qkv_ref, e_ref, w_ref, o_ref, 0)
    _head(qkv_ref, e_ref, w_ref, o_ref, 1)


def kernel(qkv, cls_embedding, W_cls):
    bs, width, T = qkv.shape
    n_heads = 16
    ch = width // (3 * n_heads)
    B = bs * n_heads
    E = cls_embedding.shape[2]
    qkv_r = qkv.reshape(B, 3 * ch, T)
    out = pl.pallas_call(
        _pab_kernel,
        grid=(B // 2,),
        in_specs=[
            pl.BlockSpec((2, 3 * ch, T), lambda b: (b, 0, 0)),
            pl.BlockSpec((2, T, E), lambda b: (b, 0, 0)),
            pl.BlockSpec((3 * ch, E), lambda b: (0, 0)),
        ],
        out_specs=pl.BlockSpec((2, ch, T), lambda b: (b, 0, 0)),
        out_shape=jax.ShapeDtypeStruct((B, ch, T), qkv.dtype),
    )(qkv_r, cls_embedding, W_cls)
    return out.reshape(bs, n_heads * ch, T)


# speedup vs baseline: 3.3150x; 1.0170x over previous
import math

import jax
import jax.numpy as jnp
from jax.experimental import pallas as pl

_NTILE = 2


def _head(qkv_ref, e_ref, w_ref, o_ref, h):
    ch = o_ref.shape[1]
    T = o_ref.shape[2]
    scale = 1.0 / math.sqrt(math.sqrt(ch))
    e = e_ref[h]
    dn_te = (((1,), (1,)), ((), ()))
    null = jax.lax.dot_general(
        w_ref[...], e, dn_te, preferred_element_type=jnp.float32)
    qe = (qkv_ref[h, 0:ch, :] + null[0:ch, :]) * scale
    ke = (qkv_ref[h, ch:2 * ch, :] + null[ch:2 * ch, :]) * scale
    ve = qkv_ref[h, 2 * ch:3 * ch, :] + null[2 * ch:3 * ch, :]
    ve_aug = jnp.concatenate(
        [ve.T, jnp.ones((T, 8), dtype=jnp.float32)], axis=1)
    ve_aug16 = ve_aug.astype(jnp.bfloat16)
    tiles = []
    tw = T // _NTILE
    for i in range(_NTILE):
        logits = jax.lax.dot_general(
            qe[:, i * tw:(i + 1) * tw], ke, (((0,), (0,)), ((), ())),
            preferred_element_type=jnp.float32)
        ew = jnp.exp(logits).astype(jnp.bfloat16)
        tiles.append(jax.lax.dot_general(
            ew, ve_aug16, (((1,), (0,)), ((), ())),
            preferred_element_type=jnp.float32))
    a_aug = jnp.concatenate(tiles, axis=0)
    a_t = a_aug[:, 0:ch] / a_aug[:, ch:ch + 1]
    o_ref[h] = a_t.T


def _pab_kernel(qkv_ref, e_ref, w_ref, o_ref):
    _head(qkv_ref, e_ref, w_ref, o_ref, 0)
    _head(qkv_ref, e_ref, w_ref, o_ref, 1)


def kernel(qkv, cls_embedding, W_cls):
    bs, width, T = qkv.shape
    n_heads = 16
    ch = width // (3 * n_heads)
    B = bs * n_heads
    E = cls_embedding.shape[2]
    qkv_r = qkv.reshape(B, 3 * ch, T)
    out = pl.pallas_call(
        _pab_kernel,
        grid=(B // 2,),
        in_specs=[
            pl.BlockSpec((2, 3 * ch, T), lambda b: (b, 0, 0)),
            pl.BlockSpec((2, T, E), lambda b: (b, 0, 0)),
            pl.BlockSpec((3 * ch, E), lambda b: (0, 0)),
        ],
        out_specs=pl.BlockSpec((2, ch, T), lambda b: (b, 0, 0)),
        out_shape=jax.ShapeDtypeStruct((B, ch, T), qkv.dtype),
    )(qkv_r, cls_embedding, W_cls)
    return out.reshape(bs, n_heads * ch, T)
